# speedup vs baseline: 1.8222x; 1.8222x over previous
import functools

import jax
import jax.numpy as jnp
from jax import lax
from jax.experimental import pallas as pl
from jax.experimental.pallas import tpu as pltpu
from jax.experimental.pallas import tpu_sc as plsc

N = 100000
D = 128
NC = 2
NS = 16
NW = NC * NS

ROWS_W = 3128
ROWS_LAST = N - (NW - 1) * ROWS_W
CHUNK = 136
NCHUNK = ROWS_W // CHUNK
TAIL = ROWS_LAST - (NCHUNK - 1) * CHUNK

_mesh = plsc.VectorSubcoreMesh(core_axis_name="c", subcore_axis_name="s")


@functools.partial(
    pl.kernel,
    out_type=jax.ShapeDtypeStruct((N, D), jnp.float32),
    mesh=_mesh,
    scratch_types=[
        pltpu.VMEM((CHUNK,), jnp.int32),
        pltpu.VMEM((CHUNK, D), jnp.float32),
        pltpu.VMEM((TAIL,), jnp.int32),
        pltpu.VMEM((TAIL, D), jnp.float32),
        pltpu.SemaphoreType.DMA,
    ],
)
def _gather_rows(src_hbm, x_hbm, out_hbm, idx_v, rows_v, idx_t, rows_t, sem):
    w = lax.axis_index("s") * NC + lax.axis_index("c")
    base = w * ROWS_W
    nfull = jnp.where(w == NW - 1, NCHUNK - 1, NCHUNK)

    @pl.loop(0, NCHUNK)
    def _(c):
        @pl.when(c < nfull)
        def _():
            off = base + c * CHUNK
            pltpu.sync_copy(src_hbm.at[pl.ds(off, CHUNK)], idx_v)
            pltpu.async_copy(x_hbm.at[idx_v], rows_v, sem).wait()
            pltpu.sync_copy(rows_v, out_hbm.at[pl.ds(off, CHUNK)])

    @pl.when(w == NW - 1)
    def _():
        off = base + (NCHUNK - 1) * CHUNK
        pltpu.sync_copy(src_hbm.at[pl.ds(off, TAIL)], idx_t)
        pltpu.async_copy(x_hbm.at[idx_t], rows_t, sem).wait()
        pltpu.sync_copy(rows_t, out_hbm.at[pl.ds(off, TAIL)])


def kernel(x, indices, shuffle_perm):
    idx = indices.astype(jnp.int32)
    perm = shuffle_perm.astype(jnp.int32)
    shuffled = jnp.take(idx, perm, axis=0)
    src = (
        jnp.arange(N, dtype=jnp.int32)
        .at[idx]
        .set(shuffled, unique_indices=True, mode="promise_in_bounds")
    )
    return _gather_rows(src, x)


# speedup vs baseline: 6.6666x; 3.6585x over previous
import functools

import jax
import jax.numpy as jnp
from jax import lax
from jax.experimental import pallas as pl
from jax.experimental.pallas import tpu as pltpu
from jax.experimental.pallas import tpu_sc as plsc

N = 100000
D = 128
M = 50000
NC = 2
NS = 16
NW = NC * NS

NP = 100096
MP = 51200
UPD_W = MP // NS
SEG = NP // NS

ROWS_W = 3128
ROWS_LAST = N - (NW - 1) * ROWS_W
CHUNK = 136
NCHUNK = ROWS_W // CHUNK
TAIL = ROWS_LAST - (NCHUNK - 1) * CHUNK

_mesh = plsc.VectorSubcoreMesh(core_axis_name="c", subcore_axis_name="s")


@functools.partial(
    pl.kernel,
    out_type=jax.ShapeDtypeStruct((N, D), jnp.float32),
    mesh=_mesh,
    scratch_types=[
        pltpu.VMEM((UPD_W,), jnp.int32),
        pltpu.VMEM((UPD_W,), jnp.int32),
        pltpu.VMEM((SEG,), jnp.int32),
        pltpu.VMEM_SHARED((NP,), jnp.int32),
        pltpu.VMEM((CHUNK,), jnp.int32),
        pltpu.VMEM((CHUNK, D), jnp.float32),
        pltpu.VMEM((TAIL,), jnp.int32),
        pltpu.VMEM((TAIL, D), jnp.float32),
        pltpu.SemaphoreType.DMA,
    ],
)
def _shuffle(upos_hbm, usrc_hbm, x_hbm, out_hbm,
             upos_v, usrc_v, iota_v, src_sh, idx_v, rows_v, idx_t, rows_t,
             sem):
    s_id = lax.axis_index("s")
    w = s_id * NC + lax.axis_index("c")

    seg_base = s_id * SEG

    @pl.loop(0, SEG // 16)
    def _(i):
        iota_v[pl.ds(i * 16, 16)] = lax.iota(jnp.int32, 16) + (
            seg_base + i * 16)

    pltpu.sync_copy(iota_v, src_sh.at[pl.ds(seg_base, SEG)])
    pltpu.sync_copy(upos_hbm.at[pl.ds(s_id * UPD_W, UPD_W)], upos_v)
    pltpu.sync_copy(usrc_hbm.at[pl.ds(s_id * UPD_W, UPD_W)], usrc_v)
    plsc.subcore_barrier()

    pltpu.sync_copy(usrc_v, src_sh.at[upos_v])
    plsc.subcore_barrier()

    base = w * ROWS_W
    nfull = jnp.where(w == NW - 1, NCHUNK - 1, NCHUNK)

    @pl.loop(0, NCHUNK)
    def _(c):
        @pl.when(c < nfull)
        def _():
            off = base + c * CHUNK
            pltpu.sync_copy(src_sh.at[pl.ds(off, CHUNK)], idx_v)
            pltpu.async_copy(x_hbm.at[idx_v], rows_v, sem).wait()
            pltpu.sync_copy(rows_v, out_hbm.at[pl.ds(off, CHUNK)])

    @pl.when(w == NW - 1)
    def _():
        off = base + (NCHUNK - 1) * CHUNK
        pltpu.sync_copy(src_sh.at[pl.ds(off, TAIL)], idx_t)
        pltpu.async_copy(x_hbm.at[idx_t], rows_t, sem).wait()
        pltpu.sync_copy(rows_t, out_hbm.at[pl.ds(off, TAIL)])


def kernel(x, indices, shuffle_perm):
    idx = indices.astype(jnp.int32)
    perm = shuffle_perm.astype(jnp.int32)
    shuffled = jnp.take(idx, perm, axis=0)
    upos = jnp.concatenate([idx, jnp.full((MP - M,), N, jnp.int32)])
    usrc = jnp.concatenate([shuffled, jnp.zeros((MP - M,), jnp.int32)])
    return _shuffle(upos, usrc, x)


# speedup vs baseline: 8.2977x; 1.2447x over previous
import functools

import jax
import jax.numpy as jnp
from jax import lax
from jax.experimental import pallas as pl
from jax.experimental.pallas import tpu as pltpu
from jax.experimental.pallas import tpu_sc as plsc

N = 100000
D = 128
M = 50000
NC = 2
NS = 16
NW = NC * NS

NP = 100096
MP = 51200
UPD_W = MP // NS
SEG = NP // NS

ROWS_W = 3128
ROWS_LAST = N - (NW - 1) * ROWS_W
CHUNK = 184
NCH = ROWS_W // CHUNK
TAIL = ROWS_LAST - (NCH - 1) * CHUNK
NBUF = 3
NSLOT = ((NCH + NBUF - 1) // NBUF) * NBUF

_mesh = plsc.VectorSubcoreMesh(core_axis_name="c", subcore_axis_name="s")


@functools.partial(
    pl.kernel,
    out_type=jax.ShapeDtypeStruct((N, D), jnp.float32),
    mesh=_mesh,
    scratch_types=[
        pltpu.VMEM((UPD_W,), jnp.int32),
        pltpu.VMEM((UPD_W,), jnp.int32),
        pltpu.VMEM((SEG,), jnp.int32),
        pltpu.VMEM_SHARED((NP,), jnp.int32),
        [pltpu.VMEM((CHUNK,), jnp.int32) for _ in range(NBUF)],
        [pltpu.VMEM((CHUNK, D), jnp.float32) for _ in range(NBUF)],
        [pltpu.SemaphoreType.DMA for _ in range(NBUF)],
        [pltpu.SemaphoreType.DMA for _ in range(NBUF)],
        pltpu.VMEM((TAIL,), jnp.int32),
        pltpu.VMEM((TAIL, D), jnp.float32),
        pltpu.SemaphoreType.DMA,
    ],
)
def _shuffle(upos_hbm, usrc_hbm, x_hbm, out_hbm,
             upos_v, usrc_v, iota_v, src_sh, idxs, bufs, gsem, wsem,
             idx_t, rows_t, sem):
    s_id = lax.axis_index("s")
    w = s_id * NC + lax.axis_index("c")

    seg_base = s_id * SEG

    @pl.loop(0, SEG // 16)
    def _(i):
        iota_v[pl.ds(i * 16, 16)] = lax.iota(jnp.int32, 16) + (
            seg_base + i * 16)

    pltpu.sync_copy(iota_v, src_sh.at[pl.ds(seg_base, SEG)])
    pltpu.sync_copy(upos_hbm.at[pl.ds(s_id * UPD_W, UPD_W)], upos_v)
    pltpu.sync_copy(usrc_hbm.at[pl.ds(s_id * UPD_W, UPD_W)], usrc_v)
    plsc.subcore_barrier()

    pltpu.sync_copy(usrc_v, src_sh.at[upos_v])
    plsc.subcore_barrier()

    base = w * ROWS_W
    nfull = jnp.where(w == NW - 1, NCH - 1, NCH)

    @pl.loop(0, NSLOT, step=NBUF)
    def _(c0):
        for b in range(NBUF):
            c = c0 + b

            @pl.when(jnp.logical_and(c >= NBUF, c < nfull))
            def _(b=b):
                pltpu.make_async_copy(
                    bufs[b], out_hbm.at[pl.ds(base, CHUNK)], wsem[b]).wait()

            @pl.when(c < nfull)
            def _(b=b, c=c):
                pltpu.sync_copy(src_sh.at[pl.ds(base + c * CHUNK, CHUNK)],
                                idxs[b])
                pltpu.async_copy(x_hbm.at[idxs[b]], bufs[b], gsem[b])

            b1 = (b - 1) % NBUF

            @pl.when(jnp.logical_and(c >= 1, c - 1 < nfull))
            def _(b1=b1, c=c):
                pltpu.make_async_copy(
                    x_hbm.at[idxs[b1]], bufs[b1], gsem[b1]).wait()
                pltpu.async_copy(
                    bufs[b1],
                    out_hbm.at[pl.ds(base + (c - 1) * CHUNK, CHUNK)],
                    wsem[b1])

    for b in range(NBUF):
        pltpu.make_async_copy(
            bufs[b], out_hbm.at[pl.ds(base, CHUNK)], wsem[b]).wait()

    @pl.when(w == NW - 1)
    def _():
        off = base + (NCH - 1) * CHUNK
        pltpu.sync_copy(src_sh.at[pl.ds(off, TAIL)], idx_t)
        pltpu.async_copy(x_hbm.at[idx_t], rows_t, sem).wait()
        pltpu.sync_copy(rows_t, out_hbm.at[pl.ds(off, TAIL)])


def kernel(x, indices, shuffle_perm):
    idx = indices.astype(jnp.int32)
    perm = shuffle_perm.astype(jnp.int32)
    shuffled = jnp.take(idx, perm, axis=0)
    upos = jnp.concatenate([idx, jnp.full((MP - M,), N, jnp.int32)])
    usrc = jnp.concatenate([shuffled, jnp.zeros((MP - M,), jnp.int32)])
    return _shuffle(upos, usrc, x)


# speedup vs baseline: 9.1246x; 1.0997x over previous
import functools

import jax
import jax.numpy as jnp
from jax import lax
from jax.experimental import pallas as pl
from jax.experimental.pallas import tpu as pltpu
from jax.experimental.pallas import tpu_sc as plsc

N = 100000
D = 128
M = 50000
NC = 2
NS = 16
NW = NC * NS

NP = 100096
UPD_W = 3128
SEG = NP // NS

ROWS_W = 3128
ROWS_LAST = N - (NW - 1) * ROWS_W
CHUNK = 184
NCH = ROWS_W // CHUNK
TAIL = ROWS_LAST - (NCH - 1) * CHUNK
NBUF = 3
NSLOT = ((NCH + NBUF - 1) // NBUF) * NBUF

_mesh = plsc.VectorSubcoreMesh(core_axis_name="c", subcore_axis_name="s")


@functools.partial(
    pl.kernel,
    out_type=jax.ShapeDtypeStruct((N, D), jnp.float32),
    mesh=_mesh,
    scratch_types=[
        pltpu.VMEM((UPD_W,), jnp.int32),
        pltpu.VMEM((UPD_W,), jnp.int32),
        pltpu.VMEM((UPD_W,), jnp.int32),
        pltpu.VMEM((SEG,), jnp.int32),
        pltpu.VMEM_SHARED((NP,), jnp.int32),
        [pltpu.VMEM((CHUNK,), jnp.int32) for _ in range(NBUF)],
        [pltpu.VMEM((CHUNK, D), jnp.float32) for _ in range(NBUF)],
        [pltpu.SemaphoreType.DMA for _ in range(NBUF)],
        [pltpu.SemaphoreType.DMA for _ in range(NBUF)],
        pltpu.VMEM((TAIL,), jnp.int32),
        pltpu.VMEM((TAIL, D), jnp.float32),
        pltpu.SemaphoreType.DMA,
    ],
)
def _shuffle(idx_hbm, perm_hbm, x_hbm, out_hbm,
             upos_v, perm_v, usrc_v, iota_v, src_sh, idxs, bufs, gsem, wsem,
             idx_t, rows_t, sem):
    s_id = lax.axis_index("s")
    w = s_id * NC + lax.axis_index("c")

    seg_base = s_id * SEG

    @pl.loop(0, SEG // 16)
    def _(i):
        iota_v[pl.ds(i * 16, 16)] = lax.iota(jnp.int32, 16) + (
            seg_base + i * 16)

    pltpu.sync_copy(iota_v, src_sh.at[pl.ds(seg_base, SEG)])

    ubase = jnp.minimum(s_id * UPD_W, M - UPD_W)
    pltpu.sync_copy(idx_hbm.at[pl.ds(ubase, UPD_W)], upos_v)
    pltpu.sync_copy(perm_hbm.at[pl.ds(ubase, UPD_W)], perm_v)
    pltpu.async_copy(idx_hbm.at[perm_v], usrc_v, sem).wait()
    plsc.subcore_barrier()

    pltpu.sync_copy(usrc_v, src_sh.at[upos_v])
    plsc.subcore_barrier()

    base = w * ROWS_W
    nfull = jnp.where(w == NW - 1, NCH - 1, NCH)

    @pl.loop(0, NSLOT, step=NBUF)
    def _(c0):
        for b in range(NBUF):
            c = c0 + b

            @pl.when(jnp.logical_and(c >= NBUF, c < nfull))
            def _(b=b):
                pltpu.make_async_copy(
                    bufs[b], out_hbm.at[pl.ds(base, CHUNK)], wsem[b]).wait()

            @pl.when(c < nfull)
            def _(b=b, c=c):
                pltpu.sync_copy(src_sh.at[pl.ds(base + c * CHUNK, CHUNK)],
                                idxs[b])
                pltpu.async_copy(x_hbm.at[idxs[b]], bufs[b], gsem[b])

            b1 = (b - 1) % NBUF

            @pl.when(jnp.logical_and(c >= 1, c - 1 < nfull))
            def _(b1=b1, c=c):
                pltpu.make_async_copy(
                    x_hbm.at[idxs[b1]], bufs[b1], gsem[b1]).wait()
                pltpu.async_copy(
                    bufs[b1],
                    out_hbm.at[pl.ds(base + (c - 1) * CHUNK, CHUNK)],
                    wsem[b1])

    for b in range(NBUF):
        pltpu.make_async_copy(
            bufs[b], out_hbm.at[pl.ds(base, CHUNK)], wsem[b]).wait()

    @pl.when(w == NW - 1)
    def _():
        off = base + (NCH - 1) * CHUNK
        pltpu.sync_copy(src_sh.at[pl.ds(off, TAIL)], idx_t)
        pltpu.async_copy(x_hbm.at[idx_t], rows_t, sem).wait()
        pltpu.sync_copy(rows_t, out_hbm.at[pl.ds(off, TAIL)])


def kernel(x, indices, shuffle_perm):
    idx = indices.astype(jnp.int32)
    perm = shuffle_perm.astype(jnp.int32)
    return _shuffle(idx, perm, x)
